# baseline (speedup 1.0000x reference)
import functools

import jax
import jax.numpy as jnp
from jax import lax
from jax.experimental import pallas as pl
from jax.experimental.pallas import tpu as pltpu
from jax.experimental.pallas import tpu_sc as plsc

D = 64
NW = 32
G = 128


@functools.cache
def _make_gather(B):
    BPW = B // NW
    NCH = BPW // G
    mesh = plsc.VectorSubcoreMesh(core_axis_name="c", subcore_axis_name="s")

    @functools.partial(
        pl.kernel,
        mesh=mesh,
        out_type=jax.ShapeDtypeStruct((B, D), jnp.float32),
        scratch_types=[
            pltpu.VMEM((BPW,), jnp.int32),
            pltpu.VMEM((2, G, D), jnp.float32),
            pltpu.SemaphoreType.DMA,
            pltpu.SemaphoreType.DMA,
        ],
        compiler_params=pltpu.CompilerParams(use_tc_tiling_on_sc=False),
    )
    def gather_k(idx_hbm, table_hbm, out_hbm, idx_v, rows_v, sem0, sem1):
        wid = lax.axis_index("s") * 2 + lax.axis_index("c")
        base = wid * BPW
        pltpu.sync_copy(idx_hbm.at[pl.ds(base, BPW)], idx_v)

        def body(t, carry):
            j0 = t * 2
            j1 = j0 + 1
            c0 = pltpu.async_copy(
                table_hbm.at[idx_v.at[pl.ds(j0 * G, G)]], rows_v.at[0], sem0)
            c1 = pltpu.async_copy(
                table_hbm.at[idx_v.at[pl.ds(j1 * G, G)]], rows_v.at[1], sem1)
            c0.wait()
            pltpu.sync_copy(rows_v.at[0], out_hbm.at[pl.ds(base + j0 * G, G)])
            c1.wait()
            pltpu.sync_copy(rows_v.at[1], out_hbm.at[pl.ds(base + j1 * G, G)])
            return carry

        lax.fori_loop(0, NCH // 2, body, 0)

    return gather_k


def kernel(x, table):
    bsz, hist = x.shape
    flat = x.reshape(bsz * hist)
    out = _make_gather(bsz * hist)(flat, table)
    return out.reshape(bsz, hist, D)


# speedup vs baseline: 1.0469x; 1.0469x over previous
import functools

import jax
import jax.numpy as jnp
from jax import lax
from jax.experimental import pallas as pl
from jax.experimental.pallas import tpu as pltpu
from jax.experimental.pallas import tpu_sc as plsc

D = 64
NW = 32
G = 128
NBUF = 8
LA = 4


@functools.cache
def _make_gather(B):
    BPW = B // NW
    NCH = BPW // G
    assert NCH % NBUF == 0 and NCH >= 2 * NBUF
    mesh = plsc.VectorSubcoreMesh(core_axis_name="c", subcore_axis_name="s")

    @functools.partial(
        pl.kernel,
        mesh=mesh,
        out_type=jax.ShapeDtypeStruct((B, D), jnp.float32),
        scratch_types=[
            pltpu.VMEM((BPW,), jnp.int32),
            pltpu.VMEM((NBUF, G, D), jnp.float32),
            pltpu.SemaphoreType.DMA((NBUF,)),
            pltpu.SemaphoreType.DMA((NBUF,)),
        ],
        compiler_params=pltpu.CompilerParams(use_tc_tiling_on_sc=False),
    )
    def gather_k(idx_hbm, table_hbm, out_hbm, idx_v, rows_v, gsem, ssem):
        wid = lax.axis_index("s") * 2 + lax.axis_index("c")
        base = wid * BPW
        pltpu.sync_copy(idx_hbm.at[pl.ds(base, BPW)], idx_v)

        def g_start(b, j):
            pltpu.make_async_copy(
                table_hbm.at[idx_v.at[pl.ds(j * G, G)]], rows_v.at[b],
                gsem.at[b]).start()

        def g_wait(b):
            pltpu.make_async_copy(
                table_hbm.at[idx_v.at[pl.ds(0, G)]], rows_v.at[b],
                gsem.at[b]).wait()

        def s_start(b, j):
            pltpu.make_async_copy(
                rows_v.at[b], out_hbm.at[pl.ds(base + j * G, G)],
                ssem.at[b]).start()

        def s_wait(b):
            pltpu.make_async_copy(
                rows_v.at[b], out_hbm.at[pl.ds(base, G)], ssem.at[b]).wait()

        for b in range(LA):
            g_start(b, b)
        for j in range(NBUF):
            b = j % NBUF
            g_wait(b)
            s_start(b, j)
            bn = (j + LA) % NBUF
            if j >= NBUF - LA:
                s_wait(bn)
            g_start(bn, j + LA)

        def body(t, carry):
            for b in range(NBUF):
                j = t * NBUF + b
                g_wait(b)
                s_start(b, j)
                bn = (b + LA) % NBUF
                s_wait(bn)
                g_start(bn, j + LA)
            return carry

        lax.fori_loop(1, NCH // NBUF - 1, body, 0)

        for j in range(NCH - NBUF, NCH):
            b = j % NBUF
            g_wait(b)
            s_start(b, j)
            if j + LA < NCH:
                bn = (j + LA) % NBUF
                s_wait(bn)
                g_start(bn, j + LA)

        for b in range(NBUF):
            s_wait(b)

    return gather_k


def kernel(x, table):
    bsz, hist = x.shape
    flat = x.reshape(bsz * hist)
    out = _make_gather(bsz * hist)(flat, table)
    return out.reshape(bsz, hist, D)
